# Initial kernel scaffold; baseline (speedup 1.0000x reference)
import functools

import jax
import jax.numpy as jnp
import numpy as np
from jax import lax
from jax.experimental import pallas as pl
from jax.experimental.pallas import tpu as pltpu
from jax.experimental.pallas import tpu_sc as plsc

INPUT_DIM = 2
LOG2_HASHMAP = 19
NUM_LEVELS = 16
F_PER_LEVEL = 2
START_RES = 16
B_SCALE = 1.447269237440378
NUM_VEC = 2 ** LOG2_HASHMAP
MASK19 = NUM_VEC - 1
PI2_I32 = np.int32(np.uint32(2654435761).view(np.int32))
RES = [int(B_SCALE ** i * START_RES) for i in range(NUM_LEVELS)]

NC = 2
NS = 16
NW = NC * NS

B = 262144
PTS_PER_W = B // NW
C = 1024
NCHUNK = PTS_PER_W // C
NGRP = C // 16
NROW = (C * 4) // 128


def _encode_body(x0_hbm, x1_hbm, tab_hbm, out_hbm,
                 idx_a, idx_b, rows_a, rows_b, x0_v, x1_v, out_v,
                 sem_a, sem_b):
    iota = lax.iota(jnp.int32, 16)
    zero16 = jnp.zeros((16,), jnp.int32)
    one16 = jnp.ones((16,), jnp.int32)
    idx_refs = (idx_a, idx_b)
    rows_refs = (rows_a, rows_b)
    sems = (sem_a, sem_b)

    wid = lax.axis_index("s") * NC + lax.axis_index("c")

    def frac_coords(g, res_f):
        p0 = x0_v[pl.ds(g * 16, 16)]
        p1 = x1_v[pl.ds(g * 16, 16)]
        xr0 = p0 * res_f
        xr1 = p1 * res_f
        i0 = xr0.astype(jnp.int32)
        i1 = xr1.astype(jnp.int32)
        d0 = xr0 - i0.astype(jnp.float32)
        d1 = xr1 - i1.astype(jnp.float32)
        return i0, i1, d0, d1

    def gen_fire(l, sel):
        res_f = np.float32(RES[l])
        loff = np.int32(l << LOG2_HASHMAP)
        idx_ref = idx_refs[sel]
        rows_ref = rows_refs[sel]
        sem = sems[sel]

        def body(t, carry):
            for k in range(2):
                g = t * 2 + k
                i0, i1, _, _ = frac_coords(g, res_f)
                b0 = i1 * PI2_I32
                b1 = b0 + PI2_I32
                a1 = i0 + 1
                h00 = ((i0 ^ b0) & MASK19) + loff
                h01 = ((i0 ^ b1) & MASK19) + loff
                h10 = ((a1 ^ b0) & MASK19) + loff
                h11 = ((a1 ^ b1) & MASK19) + loff
                s = k * 4
                idx_ref[t, pl.ds((s + 0) * 16, 16)] = h00
                idx_ref[t, pl.ds((s + 1) * 16, 16)] = h01
                idx_ref[t, pl.ds((s + 2) * 16, 16)] = h10
                idx_ref[t, pl.ds((s + 3) * 16, 16)] = h11
            pltpu.make_async_copy(
                tab_hbm.at[idx_ref.at[t]],
                rows_ref.at[pl.ds(t * 128, 128), :],
                sem,
            ).start()
            return carry

        lax.fori_loop(0, NROW, body, 0)

    def drain(sel):
        pltpu.make_async_copy(
            tab_hbm.at[pl.ds(0, C * 4)], rows_refs[sel], sems[sel]
        ).wait()

    def interp(l, sel):
        res_f = np.float32(RES[l])
        rows_ref = rows_refs[sel]
        col0 = jnp.full((16,), 2 * l, jnp.int32)
        col1 = jnp.full((16,), 2 * l + 1, jnp.int32)

        def body(t, carry):
            for k in range(2):
                g = t * 2 + k
                _, _, d0, d1 = frac_coords(g, res_f)
                r0 = g * 64 + iota
                v = []
                for c in range(4):
                    rc = r0 + (c * 16)
                    v.append((plsc.load_gather(rows_ref, [rc, zero16]),
                              plsc.load_gather(rows_ref, [rc, one16])))
                pr = g * 16 + iota
                for f, col in ((0, col0), (1, col1)):
                    c0 = v[0][f] + d0 * (v[2][f] - v[0][f])
                    c1 = v[1][f] + d0 * (v[3][f] - v[1][f])
                    cf = c0 + d1 * (c1 - c0)
                    plsc.store_scatter(out_v, [pr, col], cf)
            return carry

        lax.fori_loop(0, NROW, body, 0)

    def chunk_body(n, carry):
        base = (wid * NCHUNK + n) * C
        pltpu.sync_copy(x0_hbm.at[pl.ds(base, C)], x0_v)
        pltpu.sync_copy(x1_hbm.at[pl.ds(base, C)], x1_v)
        gen_fire(0, 0)
        for l in range(NUM_LEVELS):
            sel = l & 1
            if l + 1 < NUM_LEVELS:
                gen_fire(l + 1, 1 - sel)
            drain(sel)
            interp(l, sel)
        pltpu.sync_copy(out_v, out_hbm.at[pl.ds(base, C)])
        return carry

    lax.fori_loop(0, NCHUNK, chunk_body, 0)


@functools.partial(
    pl.kernel,
    out_type=jax.ShapeDtypeStruct((B, NUM_LEVELS * F_PER_LEVEL), jnp.float32),
    mesh=plsc.VectorSubcoreMesh(
        core_axis_name="c", subcore_axis_name="s",
        num_cores=NC, num_subcores=NS),
    scratch_types=[
        pltpu.VMEM((NROW, 128), jnp.int32),
        pltpu.VMEM((NROW, 128), jnp.int32),
        pltpu.VMEM((C * 4, 2), jnp.float32),
        pltpu.VMEM((C * 4, 2), jnp.float32),
        pltpu.VMEM((C,), jnp.float32),
        pltpu.VMEM((C,), jnp.float32),
        pltpu.VMEM((C, NUM_LEVELS * F_PER_LEVEL), jnp.float32),
        pltpu.SemaphoreType.DMA,
        pltpu.SemaphoreType.DMA,
    ],
)
def _encode(*refs):
    _encode_body(*refs)


def kernel(x, tables):
    x0 = jnp.ascontiguousarray(x[:, 0])
    x1 = jnp.ascontiguousarray(x[:, 1])
    tab = tables.reshape(NUM_LEVELS * NUM_VEC, F_PER_LEVEL)
    return _encode(x0, x1, tab)


# speedup vs baseline: 10.1654x; 10.1654x over previous
import functools

import jax
import jax.numpy as jnp
import numpy as np
from jax import lax
from jax.experimental import pallas as pl
from jax.experimental.pallas import tpu as pltpu
from jax.experimental.pallas import tpu_sc as plsc

INPUT_DIM = 2
LOG2_HASHMAP = 19
NUM_LEVELS = 16
F_PER_LEVEL = 2
START_RES = 16
B_SCALE = 1.447269237440378
NUM_VEC = 2 ** LOG2_HASHMAP
MASK19 = NUM_VEC - 1
PI2_I32 = np.int32(np.uint32(2654435761).view(np.int32))
RES = [int(B_SCALE ** i * START_RES) for i in range(NUM_LEVELS)]

NC = 2
NS = 16
NW = NC * NS

B = 262144
PTS_PER_W = B // NW
C = 1024
NCHUNK = PTS_PER_W // C
NGRP = C // 16
NROW = (C * 4) // 128


def _encode_body(x0_hbm, x1_hbm, tab_hbm, out_hbm,
                 idx_a, idx_b, rows_a, rows_b, x0_v, x1_v, out_v,
                 sem_a, sem_b):
    iota = lax.iota(jnp.int32, 16)
    zero16 = jnp.zeros((16,), jnp.int32)
    one16 = jnp.ones((16,), jnp.int32)
    idx_refs = (idx_a, idx_b)
    rows_refs = (rows_a, rows_b)
    sems = (sem_a, sem_b)

    wid = lax.axis_index("s") * NC + lax.axis_index("c")

    def frac_coords(g, res_f):
        p0 = x0_v[pl.ds(g * 16, 16)]
        p1 = x1_v[pl.ds(g * 16, 16)]
        xr0 = p0 * res_f
        xr1 = p1 * res_f
        i0 = xr0.astype(jnp.int32)
        i1 = xr1.astype(jnp.int32)
        d0 = xr0 - i0.astype(jnp.float32)
        d1 = xr1 - i1.astype(jnp.float32)
        return i0, i1, d0, d1

    def gen_fire(l, sel):
        res_f = np.float32(RES[l])
        loff = np.int32(l << LOG2_HASHMAP)
        idx_ref = idx_refs[sel]
        rows_ref = rows_refs[sel]
        sem = sems[sel]

        def body(t, carry):
            for k in range(2):
                g = t * 2 + k
                i0, i1, _, _ = frac_coords(g, res_f)
                b0 = i1 * PI2_I32
                b1 = b0 + PI2_I32
                a1 = i0 + 1
                h00 = ((i0 ^ b0) & MASK19) + loff
                h01 = ((i0 ^ b1) & MASK19) + loff
                h10 = ((a1 ^ b0) & MASK19) + loff
                h11 = ((a1 ^ b1) & MASK19) + loff
                s = k * 4
                idx_ref[t, pl.ds((s + 0) * 16, 16)] = h00
                idx_ref[t, pl.ds((s + 1) * 16, 16)] = h01
                idx_ref[t, pl.ds((s + 2) * 16, 16)] = h10
                idx_ref[t, pl.ds((s + 3) * 16, 16)] = h11
            pltpu.make_async_copy(
                tab_hbm.at[idx_ref.at[t]],
                rows_ref.at[pl.ds(t * 128, 128), :],
                sem,
            ).start()
            return carry

        lax.fori_loop(0, NROW, body, 0)

    def drain(sel):
        idx_ref = idx_refs[sel]
        rows_ref = rows_refs[sel]
        sem = sems[sel]

        def body(t, carry):
            pltpu.make_async_copy(
                tab_hbm.at[idx_ref.at[t]],
                rows_ref.at[pl.ds(t * 128, 128), :],
                sem,
            ).wait()
            return carry

        lax.fori_loop(0, NROW, body, 0)

    def interp(l, sel):
        res_f = np.float32(RES[l])
        rows_ref = rows_refs[sel]
        col0 = jnp.full((16,), 2 * l, jnp.int32)
        col1 = jnp.full((16,), 2 * l + 1, jnp.int32)

        def body(t, carry):
            for k in range(2):
                g = t * 2 + k
                _, _, d0, d1 = frac_coords(g, res_f)
                r0 = g * 64 + iota
                v = []
                for c in range(4):
                    rc = r0 + (c * 16)
                    v.append((plsc.load_gather(rows_ref, [rc, zero16]),
                              plsc.load_gather(rows_ref, [rc, one16])))
                pr = g * 16 + iota
                for f, col in ((0, col0), (1, col1)):
                    c0 = v[0][f] + d0 * (v[2][f] - v[0][f])
                    c1 = v[1][f] + d0 * (v[3][f] - v[1][f])
                    cf = c0 + d1 * (c1 - c0)
                    plsc.store_scatter(out_v, [pr, col], cf)
            return carry

        lax.fori_loop(0, NROW, body, 0)

    def chunk_body(n, carry):
        base = (wid * NCHUNK + n) * C
        pltpu.sync_copy(x0_hbm.at[pl.ds(base, C)], x0_v)
        pltpu.sync_copy(x1_hbm.at[pl.ds(base, C)], x1_v)
        gen_fire(0, 0)
        for l in range(NUM_LEVELS):
            sel = l & 1
            if l + 1 < NUM_LEVELS:
                gen_fire(l + 1, 1 - sel)
            drain(sel)
            interp(l, sel)
        pltpu.sync_copy(out_v, out_hbm.at[pl.ds(base, C)])
        return carry

    lax.fori_loop(0, NCHUNK, chunk_body, 0)


@functools.partial(
    pl.kernel,
    out_type=jax.ShapeDtypeStruct((B, NUM_LEVELS * F_PER_LEVEL), jnp.float32),
    mesh=plsc.VectorSubcoreMesh(
        core_axis_name="c", subcore_axis_name="s",
        num_cores=NC, num_subcores=NS),
    compiler_params=pltpu.CompilerParams(
        needs_layout_passes=False, use_tc_tiling_on_sc=False),
    scratch_types=[
        pltpu.VMEM((NROW, 128), jnp.int32),
        pltpu.VMEM((NROW, 128), jnp.int32),
        pltpu.VMEM((C * 4, 2), jnp.float32),
        pltpu.VMEM((C * 4, 2), jnp.float32),
        pltpu.VMEM((C,), jnp.float32),
        pltpu.VMEM((C,), jnp.float32),
        pltpu.VMEM((C, NUM_LEVELS * F_PER_LEVEL), jnp.float32),
        pltpu.SemaphoreType.DMA,
        pltpu.SemaphoreType.DMA,
    ],
)
def _encode(*refs):
    _encode_body(*refs)


def kernel(x, tables):
    x0 = x[:, 0]
    x1 = x[:, 1]
    tab = tables.reshape(NUM_LEVELS * NUM_VEC, F_PER_LEVEL)
    return _encode(x0, x1, tab)
